# Initial kernel scaffold; baseline (speedup 1.0000x reference)
import jax
import jax.numpy as jnp
from jax.experimental import pallas as pl


def kernel(x, conv1_w, conv1_b, bn_gamma, bn_beta, bn_mean, bn_var):
    raise NotImplementedError("write your pallas kernel here")


# speedup vs baseline: 1.1499x; 1.1499x over previous
import jax
import jax.numpy as jnp
from jax.experimental import pallas as pl
from jax.experimental.pallas import tpu as pltpu


def _se_gate_kernel(x_ref, w_ref, a_ref, c_ref, out_ref, *, inv_hw):
    xb = x_ref[0]
    pooled = jnp.sum(xb, axis=1, keepdims=True) * inv_hw
    conv = jnp.dot(w_ref[...], pooled,
                   preferred_element_type=jnp.float32)
    gate = jax.nn.sigmoid(a_ref[...] * conv + c_ref[...])
    out_ref[0] = (xb * gate).astype(out_ref.dtype)


def kernel(x, conv1_w, conv1_b, bn_gamma, bn_beta, bn_mean, bn_var, eps=1e-5):
    N, C, H, W = x.shape
    HW = H * W
    x2 = x.reshape(N, C, HW)

    s = bn_gamma * jax.lax.rsqrt(bn_var + eps)
    a_vec = s.reshape(C, 1).astype(jnp.float32)
    c_vec = (s * (conv1_b - bn_mean) + bn_beta).reshape(C, 1).astype(jnp.float32)
    w2 = conv1_w.reshape(C, C).astype(jnp.float32)

    block_bytes = C * HW * 4
    vmem_bytes = 4 * block_bytes + (C * C + 2 * C) * 4 + (2 << 20)
    cost = pl.CostEstimate(
        flops=int(N * (2 * C * C + 2 * C * HW)),
        transcendentals=int(N * C),
        bytes_accessed=int(2 * N * C * HW * 4 + C * C * 4),
    )
    out2 = pl.pallas_call(
        lambda xr, wr, ar, cr, orr: _se_gate_kernel(
            xr, wr, ar, cr, orr, inv_hw=1.0 / HW),
        out_shape=jax.ShapeDtypeStruct((N, C, HW), jnp.float32),
        grid=(N,),
        in_specs=[
            pl.BlockSpec((1, C, HW), lambda n: (n, 0, 0)),
            pl.BlockSpec((C, C), lambda n: (0, 0)),
            pl.BlockSpec((C, 1), lambda n: (0, 0)),
            pl.BlockSpec((C, 1), lambda n: (0, 0)),
        ],
        out_specs=pl.BlockSpec((1, C, HW), lambda n: (n, 0, 0)),
        compiler_params=pltpu.CompilerParams(
            dimension_semantics=("parallel",),
            vmem_limit_bytes=int(min(vmem_bytes, 60 << 20)),
        ),
        cost_estimate=cost,
    )(x2, w2, a_vec, c_vec)

    return out2.reshape(N, C, H, W)
